# Initial kernel scaffold; baseline (speedup 1.0000x reference)
import jax
import jax.numpy as jnp
from jax.experimental import pallas as pl


def kernel(x, edge_index, W1l, b1l, W1r, W2l, b2l, W2r):
    raise NotImplementedError("write your pallas kernel here")


# speedup vs baseline: 3.7724x; 3.7724x over previous
import functools

import jax
import jax.numpy as jnp
from jax import lax
from jax.experimental import pallas as pl
from jax.experimental.pallas import tpu as pltpu
from jax.experimental.pallas import tpu_sc as plsc

N_NODES = 10000
N_EDGES = 320000
D = 128
NP = 10240
NC = 2
NS = 16
K = 128
EP = 327680
TILE_CHUNKS = EP // K // (NC * NS)
NBLK = TILE_CHUNKS // 8
STRIPE = NP // NS


def _sc_cnt_build():
    mesh = plsc.VectorSubcoreMesh(core_axis_name="c", subcore_axis_name="s", num_cores=NC, num_subcores=NS)
    scratch = [
        pltpu.VMEM((NBLK, 8, K), jnp.int32),
        pltpu.VMEM((K,), jnp.float32),
        pltpu.VMEM((K,), jnp.float32),
        pltpu.VMEM_SHARED((NP,), jnp.float32),
    ]

    @functools.partial(
        pl.kernel, out_type=jax.ShapeDtypeStruct((NC * NP,), jnp.float32),
        mesh=mesh, scratch_types=scratch)
    def cnt_kernel(dst_hbm, cnt_out, dst_v, ones_v, zero_v, cnt_sh):
        c = lax.axis_index("c")
        s = lax.axis_index("s")
        ones16 = jnp.ones((16,), jnp.float32)
        zero16 = jnp.zeros((16,), jnp.float32)

        def fill(r, _):
            ones_v[pl.ds(r * 16, 16)] = ones16
            zero_v[pl.ds(r * 16, 16)] = zero16
            return 0
        lax.fori_loop(0, K // 16, fill, 0)
        pltpu.sync_copy(dst_hbm.at[c, s], dst_v)
        for r in range(STRIPE // K):
            pltpu.sync_copy(zero_v,
                            cnt_sh.at[pl.ds(s * STRIPE + r * K, K)])
        plsc.subcore_barrier()

        def step(q, _):
            for r in range(8):
                pltpu.sync_copy(ones_v, cnt_sh.at[dst_v.at[q, r]], add=True)
            return 0
        lax.fori_loop(0, NBLK, step, 0)
        plsc.subcore_barrier()
        pltpu.sync_copy(cnt_sh.at[pl.ds(s * STRIPE, STRIPE)],
                        cnt_out.at[pl.ds(c * NP + s * STRIPE, STRIPE)])

    return cnt_kernel


def _sc_agg_build():
    mesh = plsc.VectorSubcoreMesh(core_axis_name="c", subcore_axis_name="s", num_cores=NC, num_subcores=NS)
    scratch = [
        pltpu.VMEM((TILE_CHUNKS, K), jnp.int32),
        pltpu.VMEM((8, K), jnp.int32),
        pltpu.VMEM((8, K), jnp.int32),
        pltpu.VMEM((K, D), jnp.float32),
        pltpu.VMEM((K, D), jnp.float32),
        pltpu.SemaphoreType.DMA,
        pltpu.SemaphoreType.DMA,
        pltpu.SemaphoreType.DMA,
        pltpu.SemaphoreType.DMA,
        pltpu.VMEM_SHARED((NP, D), jnp.float32),
    ]

    @functools.partial(
        pl.kernel, out_type=jax.ShapeDtypeStruct((NC, NP, D), jnp.float32),
        mesh=mesh, scratch_types=scratch)
    def agg_kernel(feat_hbm, srcidx_hbm, dst_hbm, agg_out,
                   src_v, db0, db1, rows0, rows1,
                   sdb0, sdb1, srow0, srow1, agg_sh):
        c = lax.axis_index("c")
        s = lax.axis_index("s")
        zero16 = jnp.zeros((16,), jnp.float32)
        dbs = (db0, db1)
        sdbs = (sdb0, sdb1)
        rows = (rows0, rows1)
        srows = (srow0, srow1)

        def zero_rows(r, _):
            for q in range(D // 16):
                rows0[r, pl.ds(q * 16, 16)] = zero16
            return 0
        lax.fori_loop(0, K, zero_rows, 0)
        pltpu.sync_copy(srcidx_hbm.at[c, s], src_v)
        for r in range(STRIPE // K):
            pltpu.sync_copy(rows0, agg_sh.at[pl.ds(s * STRIPE + r * K, K)])

        pltpu.async_copy(dst_hbm.at[c, s, 0], db0, sdb0)
        pltpu.async_copy(dst_hbm.at[c, s, 1], db1, sdb1)
        pltpu.async_copy(feat_hbm.at[src_v.at[0]], rows0, srow0)
        pltpu.async_copy(feat_hbm.at[src_v.at[1]], rows1, srow1)
        plsc.subcore_barrier()

        def do_block(q, qq, prefetch_dst, max_launch_r):
            pltpu.make_async_copy(dst_hbm.at[c, s, q], dbs[qq],
                                  sdbs[qq]).wait()
            for r in range(8):
                j = q * 8 + r
                b = r % 2
                pltpu.make_async_copy(feat_hbm.at[src_v.at[j]], rows[b],
                                      srows[b]).wait()
                pltpu.sync_copy(rows[b], agg_sh.at[dbs[qq].at[r]], add=True)
                if r <= max_launch_r:
                    pltpu.async_copy(feat_hbm.at[src_v.at[j + 2]], rows[b],
                                     srows[b])
            if prefetch_dst:
                pltpu.async_copy(dst_hbm.at[c, s, q + 2], dbs[qq], sdbs[qq])

        def step(g, _):
            for qq in range(2):
                do_block(g * 2 + qq, qq, True, 7)
            return 0
        lax.fori_loop(0, NBLK // 2 - 1, step, 0)
        do_block(NBLK - 2, 0, False, 7)
        do_block(NBLK - 1, 1, False, 5)

        plsc.subcore_barrier()
        pltpu.sync_copy(agg_sh.at[pl.ds(s * STRIPE, STRIPE)],
                        agg_out.at[c, pl.ds(s * STRIPE, STRIPE)])

    return agg_kernel


_sc_cnt = _sc_cnt_build()
_sc_agg = _sc_agg_build()


def _make_tc_lin(relu: bool):
    BLK = 1000

    def body(agg_ref, cnt_ref, x_ref, wl_ref, bl_ref, wr_ref, o_ref):
        inv = 1.0 / jnp.maximum(cnt_ref[...], 1.0)
        a = (agg_ref[0] + agg_ref[1]) * inv
        dn = (((1,), (1,)), ((), ()))
        y = lax.dot_general(a, wl_ref[...], dn,
                            precision=lax.Precision.HIGHEST,
                            preferred_element_type=jnp.float32)
        y = y + lax.dot_general(x_ref[...], wr_ref[...], dn,
                                precision=lax.Precision.HIGHEST,
                                preferred_element_type=jnp.float32)
        y = y + bl_ref[...]
        if relu:
            y = jnp.maximum(y, 0.0)
        o_ref[...] = y

    return pl.pallas_call(
        body,
        grid=(N_NODES // BLK,),
        in_specs=[
            pl.BlockSpec((NC, BLK, D), lambda i: (0, i, 0)),
            pl.BlockSpec((BLK, 1), lambda i: (i, 0)),
            pl.BlockSpec((BLK, D), lambda i: (i, 0)),
            pl.BlockSpec((D, D), lambda i: (0, 0)),
            pl.BlockSpec((1, D), lambda i: (0, 0)),
            pl.BlockSpec((D, D), lambda i: (0, 0)),
        ],
        out_specs=pl.BlockSpec((BLK, D), lambda i: (i, 0)),
        out_shape=jax.ShapeDtypeStruct((N_NODES, D), jnp.float32),
    )


_tc_lin_relu = _make_tc_lin(relu=True)
_tc_lin = _make_tc_lin(relu=False)


def kernel(x, edge_index, W1l, b1l, W1r, W2l, b2l, W2r):
    src = edge_index[0].astype(jnp.int32)
    dst = edge_index[1].astype(jnp.int32)
    pad = EP - N_EDGES
    src_p = jnp.concatenate([src, jnp.zeros((pad,), jnp.int32)])
    dst_p = jnp.concatenate([dst, jnp.full((pad,), NP - 1, jnp.int32)])
    srcidx = src_p.reshape(NC, NS, TILE_CHUNKS, K)
    dstidx = dst_p.reshape(NC, NS, NBLK, 8, K)

    cnt_raw = _sc_cnt(dstidx)
    cnt = (cnt_raw[:NP] + cnt_raw[NP:]).reshape(NP, 1)
    agg1 = _sc_agg(x, srcidx, dstidx)
    h = _tc_lin_relu(agg1, cnt, x, W1l, b1l.reshape(1, D), W1r)
    agg2 = _sc_agg(h, srcidx, dstidx)
    out = _tc_lin(agg2, cnt, h, W2l, b2l.reshape(1, D), W2r)
    return out


# speedup vs baseline: 3.7986x; 1.0069x over previous
import functools

import jax
import jax.numpy as jnp
from jax import lax
from jax.experimental import pallas as pl
from jax.experimental.pallas import tpu as pltpu
from jax.experimental.pallas import tpu_sc as plsc

N_NODES = 10000
N_EDGES = 320000
D = 128
NP = 10240
NC = 2
NS = 16
K = 80
NBLK = 16
NCH = NBLK * 8
EP = NC * NS * NCH * K
STRIPE = NP // NS


def _sc_cnt_build():
    mesh = plsc.VectorSubcoreMesh(core_axis_name="c", subcore_axis_name="s",
                                  num_cores=NC, num_subcores=NS)
    scratch = [
        pltpu.VMEM((NBLK, 8, K), jnp.int32),
        pltpu.VMEM((K,), jnp.float32),
        pltpu.VMEM((K,), jnp.float32),
        pltpu.VMEM_SHARED((NP,), jnp.float32),
    ]

    @functools.partial(
        pl.kernel, out_type=jax.ShapeDtypeStruct((NC * NP,), jnp.float32),
        mesh=mesh, scratch_types=scratch)
    def cnt_kernel(dst_hbm, cnt_out, dst_v, ones_v, zero_v, cnt_sh):
        c = lax.axis_index("c")
        s = lax.axis_index("s")
        ones16 = jnp.ones((16,), jnp.float32)
        zero16 = jnp.zeros((16,), jnp.float32)

        def fill(r, _):
            ones_v[pl.ds(r * 16, 16)] = ones16
            zero_v[pl.ds(r * 16, 16)] = zero16
            return 0
        lax.fori_loop(0, K // 16, fill, 0)
        pltpu.sync_copy(dst_hbm.at[c, s], dst_v)
        for r in range(STRIPE // K):
            pltpu.sync_copy(zero_v,
                            cnt_sh.at[pl.ds(s * STRIPE + r * K, K)])
        plsc.subcore_barrier()

        def step(q, _):
            for r in range(8):
                pltpu.sync_copy(ones_v, cnt_sh.at[dst_v.at[q, r]], add=True)
            return 0
        lax.fori_loop(0, NBLK, step, 0)
        plsc.subcore_barrier()
        pltpu.sync_copy(cnt_sh.at[pl.ds(s * STRIPE, STRIPE)],
                        cnt_out.at[pl.ds(c * NP + s * STRIPE, STRIPE)])

    return cnt_kernel


def _sc_agg_build():
    mesh = plsc.VectorSubcoreMesh(core_axis_name="c", subcore_axis_name="s",
                                  num_cores=NC, num_subcores=NS)
    scratch = [
        pltpu.VMEM((8, K), jnp.int32),
        pltpu.VMEM((8, K), jnp.int32),
        pltpu.VMEM((8, K), jnp.int32),
        pltpu.VMEM((8, K), jnp.int32),
        pltpu.VMEM((K, D), jnp.float32),
        pltpu.VMEM((K, D), jnp.float32),
        pltpu.VMEM((K, D), jnp.float32),
        pltpu.VMEM((K, D), jnp.float32),
        pltpu.SemaphoreType.DMA,
        pltpu.SemaphoreType.DMA,
        pltpu.SemaphoreType.DMA,
        pltpu.SemaphoreType.DMA,
        pltpu.SemaphoreType.DMA,
        pltpu.SemaphoreType.DMA,
        pltpu.SemaphoreType.DMA,
        pltpu.SemaphoreType.DMA,
        pltpu.SemaphoreType.DMA,
        pltpu.SemaphoreType.DMA,
        pltpu.VMEM_SHARED((NP, D), jnp.float32),
    ]

    @functools.partial(
        pl.kernel, out_type=jax.ShapeDtypeStruct((NC, NP, D), jnp.float32),
        mesh=mesh, scratch_types=scratch)
    def agg_kernel(feat_hbm, srcidx_hbm, dst_hbm, agg_out,
                   sb0, sb1, db0, db1, rows0, rows1, rows2, rows3,
                   sidx0, sidx1, sg0, sg1, sg2, sg3,
                   ss0, ss1, ss2, ss3, agg_sh):
        c = lax.axis_index("c")
        s = lax.axis_index("s")
        zero16 = jnp.zeros((16,), jnp.float32)
        sb = (sb0, sb1)
        db = (db0, db1)
        rows = (rows0, rows1, rows2, rows3)
        sg = (sg0, sg1, sg2, sg3)
        ss = (ss0, ss1, ss2, ss3)
        sidx = (sidx0, sidx1)

        def zero_rows(r, _):
            for q in range(D // 16):
                rows0[r, pl.ds(q * 16, 16)] = zero16
            return 0
        lax.fori_loop(0, K, zero_rows, 0)
        pltpu.async_copy(srcidx_hbm.at[c, s, 0], sb0, sidx0)
        pltpu.async_copy(dst_hbm.at[c, s, 0], db0, sidx0)
        pltpu.async_copy(srcidx_hbm.at[c, s, 1], sb1, sidx1)
        for r in range(STRIPE // K):
            pltpu.sync_copy(rows0, agg_sh.at[pl.ds(s * STRIPE + r * K, K)])
        pltpu.make_async_copy(srcidx_hbm.at[c, s, 0], sb0, sidx0).wait()
        pltpu.make_async_copy(dst_hbm.at[c, s, 0], db0, sidx0).wait()
        pltpu.async_copy(feat_hbm.at[sb0.at[0]], rows0, sg0)
        pltpu.async_copy(feat_hbm.at[sb0.at[1]], rows1, sg1)
        plsc.subcore_barrier()

        def do_block(q, slot, first, last):
            oslot = 1 - slot
            for r in range(8):
                j = q * 8 + r
                b = r % 4
                b2 = (r + 2) % 4
                slot2 = slot if r < 6 else oslot
                r2 = (r + 2) % 8
                if r == 2 and not last:
                    pltpu.async_copy(dst_hbm.at[c, s, q + 1], db[oslot],
                                     sidx[oslot])
                if r == 5 and not last:
                    pltpu.make_async_copy(srcidx_hbm.at[c, s, q + 1],
                                          sb[oslot], sidx[oslot]).wait()
                    pltpu.make_async_copy(dst_hbm.at[c, s, q + 1],
                                          db[oslot], sidx[oslot]).wait()
                pltpu.make_async_copy(feat_hbm.at[sb[slot].at[r]], rows[b],
                                      sg[b]).wait()
                pltpu.async_copy(rows[b], agg_sh.at[db[slot].at[r]], ss[b],
                                 add=True)
                if not (first and r < 2):
                    pltpu.make_async_copy(rows[b2],
                                          agg_sh.at[db[slot].at[r]],
                                          ss[b2]).wait()
                if not (last and r >= 6):
                    pltpu.async_copy(feat_hbm.at[sb[slot2].at[r2]], rows[b2],
                                     sg[b2])
                if r == 7 and not last:
                    @pl.when(q + 2 < NBLK)
                    def _():
                        pltpu.async_copy(srcidx_hbm.at[c, s, q + 2],
                                         sb[slot], sidx[slot])

        do_block(0, 0, True, False)

        def step(g, _):
            do_block(2 * g + 1, 1, False, False)
            do_block(2 * g + 2, 0, False, False)
            return 0
        lax.fori_loop(0, (NBLK - 2) // 2, step, 0)
        do_block(NBLK - 1, 1, False, True)

        for b in (2, 3):
            pltpu.make_async_copy(rows[b], agg_sh.at[db[1].at[b]],
                                  ss[b]).wait()

        plsc.subcore_barrier()
        pltpu.sync_copy(agg_sh.at[pl.ds(s * STRIPE, STRIPE)],
                        agg_out.at[c, pl.ds(s * STRIPE, STRIPE)])

    return agg_kernel


_sc_cnt = _sc_cnt_build()
_sc_agg = _sc_agg_build()


def _make_tc_lin(relu: bool):
    BLK = 1000

    def body(agg_ref, cnt_ref, x_ref, wl_ref, bl_ref, wr_ref, o_ref):
        inv = 1.0 / jnp.maximum(cnt_ref[...], 1.0)
        a = (agg_ref[0] + agg_ref[1]) * inv
        dn = (((1,), (1,)), ((), ()))
        y = lax.dot_general(a, wl_ref[...], dn,
                            precision=lax.Precision.HIGHEST,
                            preferred_element_type=jnp.float32)
        y = y + lax.dot_general(x_ref[...], wr_ref[...], dn,
                                precision=lax.Precision.HIGHEST,
                                preferred_element_type=jnp.float32)
        y = y + bl_ref[...]
        if relu:
            y = jnp.maximum(y, 0.0)
        o_ref[...] = y

    return pl.pallas_call(
        body,
        grid=(N_NODES // BLK,),
        in_specs=[
            pl.BlockSpec((NC, BLK, D), lambda i: (0, i, 0)),
            pl.BlockSpec((BLK, 1), lambda i: (i, 0)),
            pl.BlockSpec((BLK, D), lambda i: (i, 0)),
            pl.BlockSpec((D, D), lambda i: (0, 0)),
            pl.BlockSpec((1, D), lambda i: (0, 0)),
            pl.BlockSpec((D, D), lambda i: (0, 0)),
        ],
        out_specs=pl.BlockSpec((BLK, D), lambda i: (i, 0)),
        out_shape=jax.ShapeDtypeStruct((N_NODES, D), jnp.float32),
    )


_tc_lin_relu = _make_tc_lin(relu=True)
_tc_lin = _make_tc_lin(relu=False)


def kernel(x, edge_index, W1l, b1l, W1r, W2l, b2l, W2r):
    src = edge_index[0].astype(jnp.int32)
    dst = edge_index[1].astype(jnp.int32)
    pad = EP - N_EDGES
    src_p = jnp.concatenate([src, jnp.zeros((pad,), jnp.int32)])
    dst_p = jnp.concatenate([dst, jnp.full((pad,), NP - 1, jnp.int32)])
    srcidx = src_p.reshape(NC, NS, NBLK, 8, K)
    dstidx = dst_p.reshape(NC, NS, NBLK, 8, K)

    cnt_raw = _sc_cnt(dstidx)
    cnt = (cnt_raw[:NP] + cnt_raw[NP:]).reshape(NP, 1)
    agg1 = _sc_agg(x, srcidx, dstidx)
    h = _tc_lin_relu(agg1, cnt, x, W1l, b1l.reshape(1, D), W1r)
    agg2 = _sc_agg(h, srcidx, dstidx)
    out = _tc_lin(agg2, cnt, h, W2l, b2l.reshape(1, D), W2r)
    return out


# speedup vs baseline: 3.8046x; 1.0016x over previous
import functools

import jax
import jax.numpy as jnp
from jax import lax
from jax.experimental import pallas as pl
from jax.experimental.pallas import tpu as pltpu
from jax.experimental.pallas import tpu_sc as plsc

N_NODES = 10000
N_EDGES = 320000
D = 128
NP = 10240
NC = 2
NS = 16
K = 80
NBLK = 16
NCH = NBLK * 8
EP = NC * NS * NCH * K
STRIPE = NP // NS


def _sc_cnt_build():
    mesh = plsc.VectorSubcoreMesh(core_axis_name="c", subcore_axis_name="s",
                                  num_cores=NC, num_subcores=NS)
    scratch = [
        pltpu.VMEM((NBLK, 8, K), jnp.int32),
        pltpu.VMEM((K,), jnp.float32),
        pltpu.VMEM((K,), jnp.float32),
        pltpu.VMEM_SHARED((NP,), jnp.float32),
    ]

    @functools.partial(
        pl.kernel, out_type=jax.ShapeDtypeStruct((NC * NP,), jnp.float32),
        mesh=mesh, scratch_types=scratch)
    def cnt_kernel(dst_hbm, cnt_out, dst_v, ones_v, zero_v, cnt_sh):
        c = lax.axis_index("c")
        s = lax.axis_index("s")
        ones16 = jnp.ones((16,), jnp.float32)
        zero16 = jnp.zeros((16,), jnp.float32)

        def fill(r, _):
            ones_v[pl.ds(r * 16, 16)] = ones16
            zero_v[pl.ds(r * 16, 16)] = zero16
            return 0
        lax.fori_loop(0, K // 16, fill, 0)
        pltpu.sync_copy(dst_hbm.at[c, s], dst_v)
        for r in range(STRIPE // K):
            pltpu.sync_copy(zero_v,
                            cnt_sh.at[pl.ds(s * STRIPE + r * K, K)])
        plsc.subcore_barrier()

        def step(q, _):
            for r in range(8):
                pltpu.sync_copy(ones_v, cnt_sh.at[dst_v.at[q, r]], add=True)
            return 0
        lax.fori_loop(0, NBLK, step, 0)
        plsc.subcore_barrier()
        pltpu.sync_copy(cnt_sh.at[pl.ds(s * STRIPE, STRIPE)],
                        cnt_out.at[pl.ds(c * NP + s * STRIPE, STRIPE)])

    return cnt_kernel


def _sc_agg_build():
    mesh = plsc.VectorSubcoreMesh(core_axis_name="c", subcore_axis_name="s",
                                  num_cores=NC, num_subcores=NS)
    scratch = [
        pltpu.VMEM((8, K), jnp.int32),
        pltpu.VMEM((8, K), jnp.int32),
        pltpu.VMEM((8, K), jnp.int32),
        pltpu.VMEM((8, K), jnp.int32),
        pltpu.VMEM((K, D), jnp.float32),
        pltpu.VMEM((K, D), jnp.float32),
        pltpu.VMEM((K, D), jnp.float32),
        pltpu.VMEM((K, D), jnp.float32),
        pltpu.SemaphoreType.DMA,
        pltpu.SemaphoreType.DMA,
        pltpu.SemaphoreType.DMA,
        pltpu.SemaphoreType.DMA,
        pltpu.SemaphoreType.DMA,
        pltpu.SemaphoreType.DMA,
        pltpu.SemaphoreType.DMA,
        pltpu.SemaphoreType.DMA,
        pltpu.SemaphoreType.DMA,
        pltpu.SemaphoreType.DMA,
        pltpu.VMEM_SHARED((NP, D), jnp.float32),
    ]

    @functools.partial(
        pl.kernel, out_type=jax.ShapeDtypeStruct((NC, NP, D), jnp.float32),
        mesh=mesh, scratch_types=scratch)
    def agg_kernel(feat_hbm, srcidx_hbm, dst_hbm, agg_out,
                   sb0, sb1, db0, db1, rows0, rows1, rows2, rows3,
                   sidx0, sidx1, sg0, sg1, sg2, sg3,
                   ss0, ss1, ss2, ss3, agg_sh):
        c = lax.axis_index("c")
        s = lax.axis_index("s")
        zero16 = jnp.zeros((16,), jnp.float32)
        sb = (sb0, sb1)
        db = (db0, db1)
        rows = (rows0, rows1, rows2, rows3)
        sg = (sg0, sg1, sg2, sg3)
        ss = (ss0, ss1, ss2, ss3)
        sidx = (sidx0, sidx1)

        def zero_rows(r, _):
            for q in range(D // 16):
                rows0[r, pl.ds(q * 16, 16)] = zero16
            return 0
        lax.fori_loop(0, K, zero_rows, 0)
        pltpu.async_copy(srcidx_hbm.at[c, s, 0], sb0, sidx0)
        pltpu.async_copy(dst_hbm.at[c, s, 0], db0, sidx0)
        pltpu.async_copy(srcidx_hbm.at[c, s, 1], sb1, sidx1)
        pltpu.make_async_copy(srcidx_hbm.at[c, s, 0], sb0, sidx0).wait()
        pltpu.make_async_copy(dst_hbm.at[c, s, 0], db0, sidx0).wait()
        pltpu.async_copy(feat_hbm.at[sb0.at[0]], rows2, sg2)
        pltpu.async_copy(feat_hbm.at[sb0.at[1]], rows3, sg3)
        for r in range(STRIPE // K):
            pltpu.sync_copy(rows0, agg_sh.at[pl.ds(s * STRIPE + r * K, K)])
        plsc.subcore_barrier()

        def do_block(q, slot, first, last):
            oslot = 1 - slot
            for r in range(8):
                j = q * 8 + r
                b = (r + 2) % 4
                b2 = r % 4
                slot2 = slot if r < 6 else oslot
                r2 = (r + 2) % 8
                if r == 2 and not last:
                    pltpu.async_copy(dst_hbm.at[c, s, q + 1], db[oslot],
                                     sidx[oslot])
                if r == 5 and not last:
                    pltpu.make_async_copy(srcidx_hbm.at[c, s, q + 1],
                                          sb[oslot], sidx[oslot]).wait()
                    pltpu.make_async_copy(dst_hbm.at[c, s, q + 1],
                                          db[oslot], sidx[oslot]).wait()
                pltpu.make_async_copy(feat_hbm.at[sb[slot].at[r]], rows[b],
                                      sg[b]).wait()
                pltpu.async_copy(rows[b], agg_sh.at[db[slot].at[r]], ss[b],
                                 add=True)
                if not (first and r < 2):
                    pltpu.make_async_copy(rows[b2],
                                          agg_sh.at[db[slot].at[r]],
                                          ss[b2]).wait()
                if not (last and r >= 6):
                    pltpu.async_copy(feat_hbm.at[sb[slot2].at[r2]], rows[b2],
                                     sg[b2])
                if r == 7 and not last:
                    @pl.when(q + 2 < NBLK)
                    def _():
                        pltpu.async_copy(srcidx_hbm.at[c, s, q + 2],
                                         sb[slot], sidx[slot])

        do_block(0, 0, True, False)

        def step(g, _):
            do_block(2 * g + 1, 1, False, False)
            do_block(2 * g + 2, 0, False, False)
            return 0
        lax.fori_loop(0, (NBLK - 2) // 2, step, 0)
        do_block(NBLK - 1, 1, False, True)

        for b in (0, 1):
            pltpu.make_async_copy(rows[b], agg_sh.at[db[1].at[b]],
                                  ss[b]).wait()

        plsc.subcore_barrier()
        pltpu.sync_copy(agg_sh.at[pl.ds(s * STRIPE, STRIPE)],
                        agg_out.at[c, pl.ds(s * STRIPE, STRIPE)])

    return agg_kernel


_sc_cnt = _sc_cnt_build()
_sc_agg = _sc_agg_build()


def _make_tc_lin(relu: bool):
    BLK = 1000

    def body(agg_ref, cnt_ref, x_ref, wl_ref, bl_ref, wr_ref, o_ref):
        inv = 1.0 / jnp.maximum(cnt_ref[...], 1.0)
        a = (agg_ref[0] + agg_ref[1]) * inv
        dn = (((1,), (1,)), ((), ()))
        y = lax.dot_general(a, wl_ref[...], dn,
                            precision=lax.Precision.HIGHEST,
                            preferred_element_type=jnp.float32)
        y = y + lax.dot_general(x_ref[...], wr_ref[...], dn,
                                precision=lax.Precision.HIGHEST,
                                preferred_element_type=jnp.float32)
        y = y + bl_ref[...]
        if relu:
            y = jnp.maximum(y, 0.0)
        o_ref[...] = y

    return pl.pallas_call(
        body,
        grid=(N_NODES // BLK,),
        in_specs=[
            pl.BlockSpec((NC, BLK, D), lambda i: (0, i, 0)),
            pl.BlockSpec((BLK, 1), lambda i: (i, 0)),
            pl.BlockSpec((BLK, D), lambda i: (i, 0)),
            pl.BlockSpec((D, D), lambda i: (0, 0)),
            pl.BlockSpec((1, D), lambda i: (0, 0)),
            pl.BlockSpec((D, D), lambda i: (0, 0)),
        ],
        out_specs=pl.BlockSpec((BLK, D), lambda i: (i, 0)),
        out_shape=jax.ShapeDtypeStruct((N_NODES, D), jnp.float32),
    )


_tc_lin_relu = _make_tc_lin(relu=True)
_tc_lin = _make_tc_lin(relu=False)


def kernel(x, edge_index, W1l, b1l, W1r, W2l, b2l, W2r):
    src = edge_index[0].astype(jnp.int32)
    dst = edge_index[1].astype(jnp.int32)
    pad = EP - N_EDGES
    src_p = jnp.concatenate([src, jnp.zeros((pad,), jnp.int32)])
    dst_p = jnp.concatenate([dst, jnp.full((pad,), NP - 1, jnp.int32)])
    srcidx = src_p.reshape(NC, NS, NBLK, 8, K)
    dstidx = dst_p.reshape(NC, NS, NBLK, 8, K)

    cnt_raw = _sc_cnt(dstidx)
    cnt = (cnt_raw[:NP] + cnt_raw[NP:]).reshape(NP, 1)
    agg1 = _sc_agg(x, srcidx, dstidx)
    h = _tc_lin_relu(agg1, cnt, x, W1l, b1l.reshape(1, D), W1r)
    agg2 = _sc_agg(h, srcidx, dstidx)
    out = _tc_lin(agg2, cnt, h, W2l, b2l.reshape(1, D), W2r)
    return out
